# Initial kernel scaffold; baseline (speedup 1.0000x reference)
import jax
import jax.numpy as jnp
from jax.experimental import pallas as pl


def kernel(batch, ego_feature, obs_out, Wk, bk, Wq, bq, Wv1, bv1, Wv2, bv2, g1, be1, g2, be2, W1, b1l, Wd, bd, W2, b2l):
    raise NotImplementedError("write your pallas kernel here")


# speedup vs baseline: 6.2957x; 6.2957x over previous
import functools
import jax
import jax.numpy as jnp
from jax import lax
from jax.experimental import pallas as pl
from jax.experimental.pallas import tpu as pltpu

N = 10000
E = 160000
M = 6
H = 8
D = 128
DH = D // 2
HD = D // H

BN = 80
NB = N // BN
CHUNK = 256
E_PAD = E + CHUNK


def _layer_kernel(rs_ref, ego_ref, ef_hbm, nf_hbm, bt_hbm,
                  Wq_r, bq_r, Wk_r, bk_r, Wv1_r, bv1_r, Wv2_r, bv2_r,
                  g1_r, be1_r, g2_r, be2_r, W1_r, b1_r, Wd_r, bd_r, W2_r, b2_r,
                  out_ref,
                  ef_v, nf_v, bt_v, m_s, s_s, acc, sem_ef, sem_nf, sem_bt):
    b = pl.program_id(0)
    n0 = b * BN
    s0 = rs_ref[b]
    s1 = rs_ref[b + 1]
    s0_al = (s0 // CHUNK) * CHUNK
    nchunks = (s1 - s0_al + CHUNK - 1) // CHUNK

    r_i = lax.broadcasted_iota(jnp.int32, (D, H), 0)
    c_i = lax.broadcasted_iota(jnp.int32, (D, H), 1)
    G = ((r_i // HD) == c_i).astype(jnp.float32)
    GT = G.T

    x = ego_ref[...]
    x2 = x.reshape(BN * M, D)
    mu = jnp.mean(x2, axis=-1, keepdims=True)
    var = jnp.mean((x2 - mu) ** 2, axis=-1, keepdims=True)
    ln1 = (x2 - mu) / jnp.sqrt(var + 1e-5) * g1_r[...] + be1_r[...]
    q = ln1 @ Wq_r[...] + bq_r[...]
    q3 = q.reshape(BN, M, D)

    m_s[...] = jnp.full((M, BN, H), -1e9, jnp.float32)
    s_s[...] = jnp.zeros((M, BN, H), jnp.float32)
    acc[...] = jnp.zeros((M, BN, D), jnp.float32)

    col = lax.broadcasted_iota(jnp.int32, (CHUNK, BN), 1)
    row1 = lax.broadcasted_iota(jnp.int32, (CHUNK, 1), 0)

    def load_common(c):
        off = s0_al + c * CHUNK
        cp_e = pltpu.make_async_copy(ef_hbm.at[pl.ds(off, CHUNK)], ef_v, sem_ef)
        cp_b = pltpu.make_async_copy(bt_hbm.at[pl.ds(off, CHUNK)], bt_v, sem_bt)
        cp_e.start()
        cp_b.start()
        cp_e.wait()
        cp_b.wait()
        bt = bt_v[...].reshape(CHUNK, 1)
        li = bt - n0
        g = off + row1
        valid = (g >= s0) & (g < s1)
        onehot = jnp.where((li == col) & valid, 1.0, 0.0)
        return off, onehot, valid

    def pass1(c, carry):
        _, onehot, valid = load_common(c)
        ef3 = ef_v[...]
        for m in range(M):
            km = ef3[:, m, :] @ Wk_r[...] + bk_r[...]
            qm = onehot @ q3[:, m, :]
            lg = (km * qm) @ G
            lg = jnp.where(valid, lg, -1e9)
            cand = jnp.max(
                jnp.where(onehot[:, :, None] > 0.0, lg[:, None, :], -1e9),
                axis=0)
            m_s[m] = jnp.maximum(m_s[m], cand)
        return carry

    def pass2(c, carry):
        off, onehot, valid = load_common(c)
        cp_n = pltpu.make_async_copy(nf_hbm.at[pl.ds(off, CHUNK)], nf_v, sem_nf)
        cp_n.start()
        cp_n.wait()
        ef3 = ef_v[...]
        nf3 = nf_v[...]
        for m in range(M):
            km = ef3[:, m, :] @ Wk_r[...] + bk_r[...]
            qm = onehot @ q3[:, m, :]
            lg = (km * qm) @ G
            vm = (ef3[:, m, :] @ Wv1_r[...] + bv1_r[...]
                  + nf3[:, m, :] @ Wv2_r[...] + bv2_r[...])
            mg = onehot @ m_s[m]
            e = jnp.where(valid, jnp.exp(lg - mg), 0.0)
            s_s[m] = s_s[m] + lax.dot_general(
                onehot, e, (((0,), (0,)), ((), ())))
            w = vm * (e @ GT)
            acc[m] = acc[m] + lax.dot_general(
                onehot, w, (((0,), (0,)), ((), ())))
        return carry

    lax.fori_loop(0, nchunks, pass1, 0)
    lax.fori_loop(0, nchunks, pass2, 0)

    sa_ms = []
    for m in range(M):
        s_wide = s_s[m] @ GT
        sa_ms.append(acc[m] / (s_wide + 1e-16))
    sa = jnp.stack(sa_ms, axis=1)
    ego1 = x + sa
    y2 = ego1.reshape(BN * M, D)
    mu2 = jnp.mean(y2, axis=-1, keepdims=True)
    var2 = jnp.mean((y2 - mu2) ** 2, axis=-1, keepdims=True)
    ln2 = (y2 - mu2) / jnp.sqrt(var2 + 1e-5) * g2_r[...] + be2_r[...]
    h = jnp.maximum(ln2 @ W1_r[...] + b1_r[...], 0.0)
    h = h @ Wd_r[...] + bd_r[...]
    out = y2 + (h @ W2_r[...] + b2_r[...])
    out_ref[...] = out.reshape(BN, M, D)


def _full_spec(shape):
    nd = len(shape)
    return pl.BlockSpec(shape, lambda b, *_: (0,) * nd)


@jax.jit
def kernel(batch, ego_feature, obs_out, Wk, bk, Wq, bq, Wv1, bv1, Wv2, bv2,
           g1, be1, g2, be2, W1, b1l, Wd, bd, W2, b2l):
    edge_feat = obs_out[0]
    node_feat = obs_out[1]
    pad = CHUNK
    ef_p = jnp.concatenate(
        [edge_feat, jnp.zeros((pad, M, DH), jnp.float32)], axis=0)
    nf_p = jnp.concatenate(
        [node_feat, jnp.zeros((pad, M, DH), jnp.float32)], axis=0)
    bt_p = jnp.concatenate(
        [batch, jnp.full((pad,), N, jnp.int32)], axis=0)
    rs = jnp.searchsorted(
        batch, jnp.arange(0, N + 1, BN, dtype=jnp.int32)).astype(jnp.int32)

    grid_spec = pltpu.PrefetchScalarGridSpec(
        num_scalar_prefetch=1,
        grid=(NB,),
        in_specs=[
            pl.BlockSpec((BN, M, D), lambda b, *_: (b, 0, 0)),
            pl.BlockSpec(memory_space=pl.ANY),
            pl.BlockSpec(memory_space=pl.ANY),
            pl.BlockSpec(memory_space=pl.ANY),
            _full_spec((D, D)), _full_spec((1, D)),
            _full_spec((DH, D)), _full_spec((1, D)),
            _full_spec((DH, D)), _full_spec((1, D)),
            _full_spec((DH, D)), _full_spec((1, D)),
            _full_spec((1, D)), _full_spec((1, D)),
            _full_spec((1, D)), _full_spec((1, D)),
            _full_spec((D, D)), _full_spec((1, D)),
            _full_spec((D, D)), _full_spec((1, D)),
            _full_spec((D, D)), _full_spec((1, D)),
        ],
        out_specs=pl.BlockSpec((BN, M, D), lambda b, *_: (b, 0, 0)),
        scratch_shapes=[
            pltpu.VMEM((CHUNK, M, DH), jnp.float32),
            pltpu.VMEM((CHUNK, M, DH), jnp.float32),
            pltpu.VMEM((CHUNK,), jnp.int32),
            pltpu.VMEM((M, BN, H), jnp.float32),
            pltpu.VMEM((M, BN, H), jnp.float32),
            pltpu.VMEM((M, BN, D), jnp.float32),
            pltpu.SemaphoreType.DMA,
            pltpu.SemaphoreType.DMA,
            pltpu.SemaphoreType.DMA,
        ],
    )
    fn = pl.pallas_call(
        _layer_kernel,
        grid_spec=grid_spec,
        out_shape=jax.ShapeDtypeStruct((N, M, D), jnp.float32),
    )
    ego = ego_feature
    for i in range(2):
        ego = fn(rs, ego, ef_p, nf_p, bt_p,
                 Wq[i], bq[i].reshape(1, D),
                 Wk[i], bk[i].reshape(1, D),
                 Wv1[i], bv1[i].reshape(1, D),
                 Wv2[i], bv2[i].reshape(1, D),
                 g1[i].reshape(1, D), be1[i].reshape(1, D),
                 g2[i].reshape(1, D), be2[i].reshape(1, D),
                 W1[i], b1l[i].reshape(1, D),
                 Wd[i], bd[i].reshape(1, D),
                 W2[i], b2l[i].reshape(1, D))
    return ego
